# baseline (speedup 1.0000x reference)
import jax
import jax.numpy as jnp
from jax import lax
from jax.experimental import pallas as pl
from jax.experimental.pallas import tpu as pltpu
from jax.experimental.pallas import tpu_sc as plsc

VOCAB = 13000
POS_VOCAB = 56
EMB = 50
EMB_PAD = 64
B = 4096
L = 50

NC = 2
NS = 16
NW = NC * NS

CB = 4
PB = B // NW
NCHUNK = PB // CB
SEG = 3 * L
ROWS = CB * SEG

COPY_OFFS = (0, 16, 32, EMB - 16)


def _emb_body(sent_hbm, left_hbm, right_hbm, words_hbm, pos_hbm, out_hbm,
              sent_v, left_v, right_v, rows_p, rows_c, sem):
    wid = lax.axis_index("s") * NC + lax.axis_index("c")
    base_b = wid * PB

    def chunk(c, carry):
        b0 = base_b + c * CB
        pltpu.sync_copy(sent_hbm.at[pl.ds(b0, CB)], sent_v)
        pltpu.sync_copy(left_hbm.at[pl.ds(b0, CB)], left_v)
        pltpu.sync_copy(right_hbm.at[pl.ds(b0, CB)], right_v)
        descs = []
        for i in range(CB):
            descs.append(pltpu.async_copy(
                words_hbm.at[sent_v.at[i]],
                rows_p.at[pl.ds(i * SEG, L)], sem))
            descs.append(pltpu.async_copy(
                pos_hbm.at[left_v.at[i]],
                rows_p.at[pl.ds(i * SEG + L, L)], sem))
            descs.append(pltpu.async_copy(
                pos_hbm.at[right_v.at[i]],
                rows_p.at[pl.ds(i * SEG + 2 * L, L)], sem))
        for d in descs:
            d.wait()

        for i in range(CB):
            def row_copy(j, carry2, i=i):
                src = rows_p.at[i * SEG + j]
                for k in COPY_OFFS:
                    rows_c[i, j, pl.ds(k, 16)] = src[pl.ds(k, 16)]
                return carry2

            lax.fori_loop(0, SEG, row_copy, 0)

        pltpu.sync_copy(rows_c, out_hbm.at[pl.ds(b0, CB)])
        return carry

    lax.fori_loop(0, NCHUNK, chunk, 0)


@jax.jit
def _emb_concat(sent_x, pos_left, pos_right, words_table, pos_table):
    words_p = jnp.pad(words_table, ((0, 0), (0, EMB_PAD - EMB)))
    pos_p = jnp.pad(pos_table, ((0, 0), (0, EMB_PAD - EMB)))
    k = pl.kernel(
        _emb_body,
        out_type=jax.ShapeDtypeStruct((B, SEG, EMB), jnp.float32),
        mesh=plsc.VectorSubcoreMesh(core_axis_name="c", subcore_axis_name="s"),
        scratch_types=[
            pltpu.VMEM((CB, L), jnp.int32),
            pltpu.VMEM((CB, L), jnp.int32),
            pltpu.VMEM((CB, L), jnp.int32),
            pltpu.VMEM((ROWS, EMB_PAD), jnp.float32),
            pltpu.VMEM((CB, SEG, EMB), jnp.float32),
            pltpu.SemaphoreType.DMA,
        ],
        compiler_params=pltpu.CompilerParams(
            use_tc_tiling_on_sc=False, needs_layout_passes=False),
    )
    return k(sent_x, pos_left, pos_right, words_p, pos_p)


def kernel(sent_x, pos_left, pos_right, y, words_table, pos_table):
    del y
    return _emb_concat(sent_x, pos_left, pos_right, words_table, pos_table)


# speedup vs baseline: 1.1257x; 1.1257x over previous
import jax
import jax.numpy as jnp
from jax import lax
from jax.experimental import pallas as pl
from jax.experimental.pallas import tpu as pltpu
from jax.experimental.pallas import tpu_sc as plsc

VOCAB = 13000
POS_VOCAB = 56
EMB = 50
EMB_PAD = 64
B = 4096
L = 50

NC = 2
NS = 16
NW = NC * NS

CB = 4
PB = B // NW
NCHUNK = PB // CB
SEG = 3 * L
ROWS = CB * SEG


def _emb_body(sent_hbm, left_hbm, right_hbm, words_hbm, pos_hbm, out_hbm,
              idx_v, rows_p0, rows_p1, rows_c, sem_g0, sem_g1, sem_w):
    wid = lax.axis_index("s") * NC + lax.axis_index("c")
    base_b = wid * PB

    pltpu.sync_copy(sent_hbm.at[pl.ds(base_b, PB)], idx_v.at[0])
    pltpu.sync_copy(left_hbm.at[pl.ds(base_b, PB)], idx_v.at[1])
    pltpu.sync_copy(right_hbm.at[pl.ds(base_b, PB)], idx_v.at[2])

    def fire(c, rows_p, sem):
        descs = []
        for i in range(CB):
            descs.append(pltpu.async_copy(
                words_hbm.at[idx_v.at[0, c * CB + i]],
                rows_p.at[pl.ds(i * SEG, L)], sem))
            descs.append(pltpu.async_copy(
                pos_hbm.at[idx_v.at[1, c * CB + i]],
                rows_p.at[pl.ds(i * SEG + L, L)], sem))
            descs.append(pltpu.async_copy(
                pos_hbm.at[idx_v.at[2, c * CB + i]],
                rows_p.at[pl.ds(i * SEG + 2 * L, L)], sem))
        return descs

    def drain(c, rows_p, sem):
        for d in fire_descs(rows_p, sem):
            d.wait()

    def fire_descs(rows_p, sem):
        descs = []
        for i in range(CB):
            descs.append(pltpu.make_async_copy(
                words_hbm.at[idx_v.at[0, i]],
                rows_p.at[pl.ds(i * SEG, L)], sem))
            descs.append(pltpu.make_async_copy(
                pos_hbm.at[idx_v.at[1, i]],
                rows_p.at[pl.ds(i * SEG + L, L)], sem))
            descs.append(pltpu.make_async_copy(
                pos_hbm.at[idx_v.at[2, i]],
                rows_p.at[pl.ds(i * SEG + 2 * L, L)], sem))
        return descs

    lane = lax.iota(jnp.int32, 16)
    tail_r = lane >> 1
    tail_c = EMB - 2 + (lane & 1)
    tail_dst = tail_r * EMB + tail_c

    def compact(rows_p):
        def group(g, carry2):
            r0 = g * 8
            dst0 = r0 * EMB
            for i in range(8):
                row = rows_p.at[r0 + i]
                for k in range(3):
                    rows_c[pl.ds(dst0 + i * EMB + 16 * k, 16)] = (
                        row[pl.ds(16 * k, 16)])
            tv = plsc.load_gather(rows_p, [tail_r + r0, tail_c])
            plsc.store_scatter(rows_c, [tail_dst + dst0], tv)
            return carry2

        lax.fori_loop(0, ROWS // 8, group, 0)

    def write(c):
        b0 = base_b + c * CB
        return pltpu.async_copy(
            rows_c, out_hbm.at[pl.ds(b0 * SEG * EMB, ROWS * EMB)], sem_w)

    def wait_write(c):
        b0 = base_b + c * CB
        pltpu.make_async_copy(
            rows_c, out_hbm.at[pl.ds(b0 * SEG * EMB, ROWS * EMB)],
            sem_w).wait()

    bufs = ((rows_p0, sem_g0), (rows_p1, sem_g1))

    fire(0, rows_p0, sem_g0)

    def step(h, carry):
        for par in range(2):
            c = 2 * h + par
            rows_p, sem_g = bufs[par]
            nrows_p, nsem_g = bufs[1 - par]
            cn = jnp.where(c + 1 < NCHUNK, c + 1, 0)
            fire(cn, nrows_p, nsem_g)
            drain(c, rows_p, sem_g)

            @pl.when(c > 0)
            def _():
                wait_write(c - 1)

            compact(rows_p)
            write(c)
        return carry

    lax.fori_loop(0, NCHUNK // 2, step, 0)
    wait_write(NCHUNK - 1)
    drain(0, rows_p0, sem_g0)


@jax.jit
def _emb_concat(sent_x, pos_left, pos_right, words_table, pos_table):
    words_p = jnp.pad(words_table, ((0, 0), (0, EMB_PAD - EMB)))
    pos_p = jnp.pad(pos_table, ((0, 0), (0, EMB_PAD - EMB)))
    k = pl.kernel(
        _emb_body,
        out_type=jax.ShapeDtypeStruct((B * SEG * EMB,), jnp.float32),
        mesh=plsc.VectorSubcoreMesh(core_axis_name="c", subcore_axis_name="s"),
        scratch_types=[
            pltpu.VMEM((3, PB, L), jnp.int32),
            pltpu.VMEM((ROWS, EMB_PAD), jnp.float32),
            pltpu.VMEM((ROWS, EMB_PAD), jnp.float32),
            pltpu.VMEM((ROWS * EMB,), jnp.float32),
            pltpu.SemaphoreType.DMA,
            pltpu.SemaphoreType.DMA,
            pltpu.SemaphoreType.DMA,
        ],
        compiler_params=pltpu.CompilerParams(
            use_tc_tiling_on_sc=False, needs_layout_passes=False),
    )
    out = k(sent_x, pos_left, pos_right, words_p, pos_p)
    return out.reshape(B, SEG, EMB)


def kernel(sent_x, pos_left, pos_right, y, words_table, pos_table):
    del y
    return _emb_concat(sent_x, pos_left, pos_right, words_table, pos_table)


# speedup vs baseline: 1.2982x; 1.1532x over previous
import jax
import jax.numpy as jnp
from jax import lax
from jax.experimental import pallas as pl
from jax.experimental.pallas import tpu as pltpu
from jax.experimental.pallas import tpu_sc as plsc

VOCAB = 13000
POS_VOCAB = 56
EMB = 50
EMB_PAD = 64
B = 4096
L = 50

NC = 2
NS = 16
NW = NC * NS

CB = 4
PB = B // NW
NCHUNK = PB // CB
SEG = 3 * L
WROWS = CB * L
CELEM = CB * SEG * EMB

GROUP_STARTS = (0, 8, 16, 24, 32, 40, 42)


def _emb_body(sent_hbm, left_hbm, right_hbm, words_hbm, pos_hbm, out_hbm,
              idx_v, pos_v, rows_p0, rows_p1, rows_c, sem_g0, sem_g1, sem_w):
    wid = lax.axis_index("s") * NC + lax.axis_index("c")
    base_b = wid * PB

    pltpu.sync_copy(sent_hbm.at[pl.ds(base_b, PB)], idx_v.at[0])
    pltpu.sync_copy(left_hbm.at[pl.ds(base_b, PB)], idx_v.at[1])
    pltpu.sync_copy(right_hbm.at[pl.ds(base_b, PB)], idx_v.at[2])
    pltpu.sync_copy(pos_hbm, pos_v)

    def fire(c, rows_p, sem):
        for i in range(CB):
            pltpu.async_copy(
                words_hbm.at[idx_v.at[0, c * CB + i]],
                rows_p.at[pl.ds(i * L, L)], sem)

    def drain(rows_p, sem):
        for i in range(CB):
            pltpu.make_async_copy(
                words_hbm.at[idx_v.at[0, i]],
                rows_p.at[pl.ds(i * L, L)], sem).wait()

    lane = lax.iota(jnp.int32, 16)
    zeros16 = jnp.zeros((16,), jnp.int32)
    tail_r = lane >> 1
    tail_c = EMB - 2 + (lane & 1)
    tail_dst = tail_r * EMB + tail_c

    def compact_words(rows_p):
        for i in range(CB):
            for j0 in GROUP_STARTS:
                r0 = i * L + j0
                dst0 = i * SEG * EMB + j0 * EMB
                for jj in range(8):
                    row = rows_p.at[r0 + jj]
                    for k in range(3):
                        rows_c[pl.ds(dst0 + jj * EMB + 16 * k, 16)] = (
                            row[pl.ds(16 * k, 16)])
                tv = plsc.load_gather(rows_p, [tail_r + r0, tail_c])
                plsc.store_scatter(rows_c, [tail_dst + dst0], tv)

    def fill_pos(c):
        for i in range(CB):
            for seg in (1, 2):
                base = i * SEG * EMB + seg * L * EMB

                def prow(j, carry, i=i, seg=seg, base=base):
                    q = plsc.load_gather(
                        idx_v, [zeros16 + seg, zeros16 + (c * CB + i),
                                zeros16 + j])
                    dst0 = base + j * EMB
                    for k in range(3):
                        v = plsc.load_gather(pos_v, [q, lane + 16 * k])
                        rows_c[pl.ds(dst0 + 16 * k, 16)] = v
                    v = plsc.load_gather(pos_v, [q, lane + (EMB - 16)])
                    rows_c[pl.ds(dst0 + EMB - 16, 16)] = v
                    return carry

                lax.fori_loop(0, L, prow, 0)

    def write(c):
        b0 = base_b + c * CB
        return pltpu.async_copy(
            rows_c, out_hbm.at[pl.ds(b0 * SEG * EMB, CELEM)], sem_w)

    def wait_write(c):
        b0 = base_b + c * CB
        pltpu.make_async_copy(
            rows_c, out_hbm.at[pl.ds(b0 * SEG * EMB, CELEM)], sem_w).wait()

    bufs = ((rows_p0, sem_g0), (rows_p1, sem_g1))
    fire(0, rows_p0, sem_g0)

    def step(h, carry):
        for par in range(2):
            c = 2 * h + par
            rows_p, sem_g = bufs[par]
            nrows_p, nsem_g = bufs[1 - par]
            cn = jnp.where(c + 1 < NCHUNK, c + 1, 0)
            fire(cn, nrows_p, nsem_g)
            drain(rows_p, sem_g)

            @pl.when(c > 0)
            def _():
                wait_write(c - 1)

            compact_words(rows_p)
            fill_pos(c)
            write(c)
        return carry

    lax.fori_loop(0, NCHUNK // 2, step, 0)
    wait_write(NCHUNK - 1)
    drain(rows_p0, sem_g0)


@jax.jit
def _emb_concat(sent_x, pos_left, pos_right, words_table, pos_table):
    words_p = jnp.pad(words_table, ((0, 0), (0, EMB_PAD - EMB)))
    pos_p = jnp.pad(pos_table, ((0, 0), (0, EMB_PAD - EMB)))
    k = pl.kernel(
        _emb_body,
        out_type=jax.ShapeDtypeStruct((B * SEG * EMB,), jnp.float32),
        mesh=plsc.VectorSubcoreMesh(core_axis_name="c", subcore_axis_name="s"),
        scratch_types=[
            pltpu.VMEM((3, PB, L), jnp.int32),
            pltpu.VMEM((POS_VOCAB, EMB_PAD), jnp.float32),
            pltpu.VMEM((WROWS, EMB_PAD), jnp.float32),
            pltpu.VMEM((WROWS, EMB_PAD), jnp.float32),
            pltpu.VMEM((CELEM,), jnp.float32),
            pltpu.SemaphoreType.DMA,
            pltpu.SemaphoreType.DMA,
            pltpu.SemaphoreType.DMA,
        ],
        compiler_params=pltpu.CompilerParams(
            use_tc_tiling_on_sc=False, needs_layout_passes=False),
    )
    out = k(sent_x, pos_left, pos_right, words_p, pos_p)
    return out.reshape(B, SEG, EMB)


def kernel(sent_x, pos_left, pos_right, y, words_table, pos_table):
    del y
    return _emb_concat(sent_x, pos_left, pos_right, words_table, pos_table)


# speedup vs baseline: 1.2992x; 1.0008x over previous
import jax
import jax.numpy as jnp
from jax import lax
from jax.experimental import pallas as pl
from jax.experimental.pallas import tpu as pltpu
from jax.experimental.pallas import tpu_sc as plsc

VOCAB = 13000
POS_VOCAB = 56
EMB = 50
EMB_PAD = 64
B = 4096
L = 50

NC = 2
NS = 16
NW = NC * NS

CB = 4
PB = B // NW
NCHUNK = PB // CB
SEG = 3 * L
WROWS = CB * L
CELEM = CB * SEG * EMB

GROUP_STARTS = (0, 8, 16, 24, 32, 40, 42)


def _emb_body(sent_hbm, left_hbm, right_hbm, words_hbm, pos_hbm, out_hbm,
              idx_v, pos_v, rows_p0, rows_p1, rows_c, sem_g0, sem_g1, sem_w):
    wid = lax.axis_index("s") * NC + lax.axis_index("c")
    base_b = wid * PB

    pltpu.sync_copy(sent_hbm.at[pl.ds(base_b, PB)], idx_v.at[0])
    pltpu.sync_copy(left_hbm.at[pl.ds(base_b, PB)], idx_v.at[1])
    pltpu.sync_copy(right_hbm.at[pl.ds(base_b, PB)], idx_v.at[2])
    pltpu.sync_copy(pos_hbm, pos_v)

    def fire(c, rows_p, sem):
        for i in range(CB):
            pltpu.async_copy(
                words_hbm.at[idx_v.at[0, c * CB + i]],
                rows_p.at[pl.ds(i * L, L)], sem)

    def drain(rows_p, sem):
        for i in range(CB):
            pltpu.make_async_copy(
                words_hbm.at[idx_v.at[0, i]],
                rows_p.at[pl.ds(i * L, L)], sem).wait()

    lane = lax.iota(jnp.int32, 16)
    zeros16 = jnp.zeros((16,), jnp.int32)
    tail_r = lane >> 1
    tail_c = EMB - 2 + (lane & 1)
    tail_dst = tail_r * EMB + tail_c

    def compact_words(rows_p):
        for i in range(CB):
            for j0 in GROUP_STARTS:
                r0 = i * L + j0
                dst0 = i * SEG * EMB + j0 * EMB
                for jj in range(8):
                    row = rows_p.at[r0 + jj]
                    for k in range(3):
                        rows_c[pl.ds(dst0 + jj * EMB + 16 * k, 16)] = (
                            row[pl.ds(16 * k, 16)])
                tv = plsc.load_gather(rows_p, [tail_r + r0, tail_c])
                plsc.store_scatter(rows_c, [tail_dst + dst0], tv)

    seg_splats = (zeros16 + 1, zeros16 + 2)
    cols = (lane, lane + 16, lane + 32, lane + (EMB - 16))
    col_dst = (0, 16, 32, EMB - 16)

    def fill_pos(c):
        for i in range(CB):
            row_splat = zeros16 + (c * CB + i)
            for seg in (1, 2):
                base = i * SEG * EMB + seg * L * EMB
                seg_splat = seg_splats[seg - 1]

                def prow(jh, carry, row_splat=row_splat, base=base,
                         seg_splat=seg_splat):
                    for u in range(2):
                        j = jh * 2 + u
                        q = plsc.load_gather(
                            idx_v, [seg_splat, row_splat, zeros16 + j])
                        dst0 = base + j * EMB
                        for ck, dk in zip(cols, col_dst):
                            v = plsc.load_gather(pos_v, [q, ck])
                            rows_c[pl.ds(dst0 + dk, 16)] = v
                    return carry

                lax.fori_loop(0, L // 2, prow, 0)

    def write(c):
        b0 = base_b + c * CB
        return pltpu.async_copy(
            rows_c, out_hbm.at[pl.ds(b0 * SEG * EMB, CELEM)], sem_w)

    def wait_write(c):
        b0 = base_b + c * CB
        pltpu.make_async_copy(
            rows_c, out_hbm.at[pl.ds(b0 * SEG * EMB, CELEM)], sem_w).wait()

    bufs = ((rows_p0, sem_g0), (rows_p1, sem_g1))
    fire(0, rows_p0, sem_g0)

    def step(h, carry):
        for par in range(2):
            c = 2 * h + par
            rows_p, sem_g = bufs[par]
            nrows_p, nsem_g = bufs[1 - par]
            cn = jnp.where(c + 1 < NCHUNK, c + 1, 0)
            fire(cn, nrows_p, nsem_g)
            drain(rows_p, sem_g)

            @pl.when(c > 0)
            def _():
                wait_write(c - 1)

            compact_words(rows_p)
            fill_pos(c)
            write(c)
        return carry

    lax.fori_loop(0, NCHUNK // 2, step, 0)
    wait_write(NCHUNK - 1)
    drain(rows_p0, sem_g0)


@jax.jit
def _emb_concat(sent_x, pos_left, pos_right, words_table, pos_table):
    words_p = jnp.pad(words_table, ((0, 0), (0, EMB_PAD - EMB)))
    pos_p = jnp.pad(pos_table, ((0, 0), (0, EMB_PAD - EMB)))
    k = pl.kernel(
        _emb_body,
        out_type=jax.ShapeDtypeStruct((B * SEG * EMB,), jnp.float32),
        mesh=plsc.VectorSubcoreMesh(core_axis_name="c", subcore_axis_name="s"),
        scratch_types=[
            pltpu.VMEM((3, PB, L), jnp.int32),
            pltpu.VMEM((POS_VOCAB, EMB_PAD), jnp.float32),
            pltpu.VMEM((WROWS, EMB_PAD), jnp.float32),
            pltpu.VMEM((WROWS, EMB_PAD), jnp.float32),
            pltpu.VMEM((CELEM,), jnp.float32),
            pltpu.SemaphoreType.DMA,
            pltpu.SemaphoreType.DMA,
            pltpu.SemaphoreType.DMA,
        ],
        compiler_params=pltpu.CompilerParams(
            use_tc_tiling_on_sc=False, needs_layout_passes=False),
    )
    out = k(sent_x, pos_left, pos_right, words_p, pos_p)
    return out.reshape(B, SEG, EMB)


def kernel(sent_x, pos_left, pos_right, y, words_table, pos_table):
    del y
    return _emb_concat(sent_x, pos_left, pos_right, words_table, pos_table)
